# Initial kernel scaffold; baseline (speedup 1.0000x reference)
import jax
import jax.numpy as jnp
from jax.experimental import pallas as pl


def kernel(x, edge_index, edge_values, W0, W1):
    raise NotImplementedError("write your pallas kernel here")


# speedup vs baseline: 14.8163x; 14.8163x over previous
import functools

import jax
import jax.numpy as jnp
from jax import lax
from jax.experimental import pallas as pl
from jax.experimental.pallas import tpu as pltpu
from jax.experimental.pallas import tpu_sc as plsc

N = 10000
E = 320000
D = 128
NC = 2
NS = 16
NW = NC * NS
EP_TILE = 10240
EP = EP_TILE * NW
KW = 128
NWIN = EP_TILE // KW
NP = 10240
NACC = NP
ZCH = NACC // NS
NROW_T = NP // NS
BN = 1024

def _mesh():
    return plsc.VectorSubcoreMesh(
        core_axis_name="c", subcore_axis_name="s",
        num_cores=NC, num_subcores=NS)


def _rowsum_body(row_hbm, ev_hbm, z_hbm, out_hbm, row_all, ev_all, row_buf,
                 ev_buf, acc, sem):
    c = lax.axis_index("c")
    s = lax.axis_index("s")
    wid = c * NS + s
    base = wid * EP_TILE
    pltpu.sync_copy(z_hbm.at[pl.ds(s * ZCH, ZCH)], acc.at[pl.ds(s * ZCH, ZCH)])
    plsc.subcore_barrier()
    pltpu.async_copy(row_hbm.at[pl.ds(base, EP_TILE)], row_all, sem).wait()
    pltpu.async_copy(ev_hbm.at[pl.ds(base, EP_TILE)], ev_all, sem).wait()

    def win(w, carry):
        o = w * KW
        for v in range(KW // 16):
            row_buf[pl.ds(v * 16, 16)] = row_all[pl.ds(o + v * 16, 16)]
            ev_buf[pl.ds(v * 16, 16)] = ev_all[pl.ds(o + v * 16, 16)]
        pltpu.sync_copy(ev_buf, acc.at[row_buf], add=True)
        return carry

    lax.fori_loop(0, NWIN, win, 0)
    plsc.subcore_barrier()
    pltpu.sync_copy(acc.at[pl.ds(s * ZCH, ZCH)],
                    out_hbm.at[c, pl.ds(s * ZCH, ZCH)])


@functools.cache
def _rowsum_call():
    return pl.kernel(
        _rowsum_body,
        out_type=jax.ShapeDtypeStruct((NC, NACC), jnp.float32),
        mesh=_mesh(),
        scratch_types=[
            pltpu.VMEM((EP_TILE,), jnp.int32),
            pltpu.VMEM((EP_TILE,), jnp.float32),
            pltpu.VMEM((KW,), jnp.int32),
            pltpu.VMEM((KW,), jnp.float32),
            pltpu.VMEM_SHARED((NACC,), jnp.float32),
            pltpu.SemaphoreType.DMA,
        ],
    )


def _spmm_body(g_hbm, row_hbm, col_hbm, ev_hbm, z_hbm, out_hbm, row_all,
               col_all, ev_all, row_buf, col_buf, ev_buf, rows_v, acc, sem):
    c = lax.axis_index("c")
    s = lax.axis_index("s")
    wid = c * NS + s
    base = wid * EP_TILE
    pltpu.sync_copy(z_hbm.at[pl.ds(s * NROW_T, NROW_T)],
                    acc.at[pl.ds(s * NROW_T, NROW_T)])
    plsc.subcore_barrier()
    pltpu.async_copy(row_hbm.at[pl.ds(base, EP_TILE)], row_all, sem).wait()
    pltpu.async_copy(col_hbm.at[pl.ds(base, EP_TILE)], col_all, sem).wait()
    pltpu.async_copy(ev_hbm.at[pl.ds(base, EP_TILE)], ev_all, sem).wait()

    def win(w, carry):
        o = w * KW
        for v in range(KW // 16):
            row_buf[pl.ds(v * 16, 16)] = row_all[pl.ds(o + v * 16, 16)]
            col_buf[pl.ds(v * 16, 16)] = col_all[pl.ds(o + v * 16, 16)]
            ev_buf[pl.ds(v * 16, 16)] = ev_all[pl.ds(o + v * 16, 16)]
        pltpu.async_copy(g_hbm.at[col_buf], rows_v, sem).wait()

        def scale16(e16, carry2):
            e0 = e16 * 16
            ew16 = ev_buf[pl.ds(e0, 16)]
            for j in range(16):
                b = jnp.full((16,), ew16[j], jnp.float32)
                for f in range(D // 16):
                    rows_v[e0 + j, pl.ds(f * 16, 16)] = (
                        rows_v[e0 + j, pl.ds(f * 16, 16)] * b)
            return carry2

        lax.fori_loop(0, KW // 16, scale16, 0)
        pltpu.sync_copy(rows_v, acc.at[row_buf], add=True)
        return carry

    lax.fori_loop(0, NWIN, win, 0)
    plsc.subcore_barrier()
    pltpu.sync_copy(acc.at[pl.ds(s * NROW_T, NROW_T)],
                    out_hbm.at[c, pl.ds(s * NROW_T, NROW_T)])


@functools.cache
def _spmm_call():
    return pl.kernel(
        _spmm_body,
        out_type=jax.ShapeDtypeStruct((NC, NP, D), jnp.float32),
        mesh=_mesh(),
        scratch_types=[
            pltpu.VMEM((EP_TILE,), jnp.int32),
            pltpu.VMEM((EP_TILE,), jnp.int32),
            pltpu.VMEM((EP_TILE,), jnp.float32),
            pltpu.VMEM((KW,), jnp.int32),
            pltpu.VMEM((KW,), jnp.int32),
            pltpu.VMEM((KW,), jnp.float32),
            pltpu.VMEM((KW, D), jnp.float32),
            pltpu.VMEM_SHARED((NP, D), jnp.float32),
            pltpu.SemaphoreType.DMA,
        ],
    )


def _dvec(rsp_ref):
    rs = rsp_ref[0, :] + rsp_ref[1, :] + 1e-6
    return jnp.clip(lax.rsqrt(rs), 0.0, 10.0)


def _k2_body(rsp_ref, x_ref, w0_ref, g1_ref):
    dv = _dvec(rsp_ref)
    xw = jnp.dot(x_ref[...], w0_ref[...], preferred_element_type=jnp.float32)
    g1_ref[...] = dv[:, None] * xw


def _k5_body(rsp_ref, hp_ref, w1_ref, g2_ref):
    dv = _dvec(rsp_ref)
    h = jax.nn.relu(dv[:, None] * (hp_ref[0] + hp_ref[1]))
    hw = jnp.dot(h, w1_ref[...], preferred_element_type=jnp.float32)
    g2_ref[...] = dv[:, None] * hw


def _k6_body(rsp_ref, op_ref, out_ref):
    dv = _dvec(rsp_ref)
    out_ref[...] = dv[:, None] * (op_ref[0] + op_ref[1])


_rsp_spec = pl.BlockSpec((NC, BN), lambda i: (0, i))
_mat_spec = pl.BlockSpec((BN, D), lambda i: (i, 0))
_par_spec = pl.BlockSpec((NC, BN, D), lambda i: (0, i, 0))
_w_spec = pl.BlockSpec((D, D), lambda i: (0, 0))

_k2_call = pl.pallas_call(
    _k2_body,
    grid=(NP // BN,),
    in_specs=[_rsp_spec, _mat_spec, _w_spec],
    out_specs=_mat_spec,
    out_shape=jax.ShapeDtypeStruct((NP, D), jnp.float32),
)

_k5_call = pl.pallas_call(
    _k5_body,
    grid=(NP // BN,),
    in_specs=[_rsp_spec, _par_spec, _w_spec],
    out_specs=_mat_spec,
    out_shape=jax.ShapeDtypeStruct((NP, D), jnp.float32),
)

_k6_call = pl.pallas_call(
    _k6_body,
    grid=(NP // BN,),
    in_specs=[_rsp_spec, _par_spec],
    out_specs=_mat_spec,
    out_shape=jax.ShapeDtypeStruct((NP, D), jnp.float32),
)


def kernel(x, edge_index, edge_values, W0, W1):
    row = edge_index[0]
    col = edge_index[1]
    pad = EP - E
    pad_idx = (jnp.arange(pad, dtype=jnp.int32) % N)
    row_p = jnp.concatenate([row, pad_idx])
    col_p = jnp.concatenate([col, pad_idx])
    ev_p = jnp.concatenate([edge_values, jnp.zeros((pad,), jnp.float32)])
    z1 = jnp.zeros((NACC,), jnp.float32)
    z2 = jnp.zeros((NP, D), jnp.float32)
    x_p = jnp.concatenate([x, jnp.zeros((NP - N, D), jnp.float32)])

    rsp = _rowsum_call()(row_p, ev_p, z1)
    g1 = _k2_call(rsp, x_p, W0)
    hp = _spmm_call()(g1, row_p, col_p, ev_p, z2)
    g2 = _k5_call(rsp, hp, W1)
    op = _spmm_call()(g2, row_p, col_p, ev_p, z2)
    return _k6_call(rsp, op)[:N]


# speedup vs baseline: 19.1790x; 1.2945x over previous
import functools

import jax
import jax.numpy as jnp
from jax import lax
from jax.experimental import pallas as pl
from jax.experimental.pallas import tpu as pltpu
from jax.experimental.pallas import tpu_sc as plsc

N = 10000
E = 320000
D = 128
NC = 2
NS = 16
NW = NC * NS
EP_TILE = 10240
EP = EP_TILE * NW
KW = 128
NWIN = EP_TILE // KW
NP = 10240
NACC = NP
ZCH = NACC // NS
NROW_T = NP // NS
BN = 1024

def _mesh():
    return plsc.VectorSubcoreMesh(
        core_axis_name="c", subcore_axis_name="s",
        num_cores=NC, num_subcores=NS)


def _rowsum_body(row_hbm, ev_hbm, z_hbm, out_hbm, row_all, ev_all, row_buf,
                 ev_buf, acc, sem):
    c = lax.axis_index("c")
    s = lax.axis_index("s")
    wid = c * NS + s
    base = wid * EP_TILE
    pltpu.sync_copy(z_hbm.at[pl.ds(s * ZCH, ZCH)], acc.at[pl.ds(s * ZCH, ZCH)])
    plsc.subcore_barrier()
    pltpu.async_copy(row_hbm.at[pl.ds(base, EP_TILE)], row_all, sem).wait()
    pltpu.async_copy(ev_hbm.at[pl.ds(base, EP_TILE)], ev_all, sem).wait()

    def win(w, carry):
        o = w * KW
        for v in range(KW // 16):
            row_buf[pl.ds(v * 16, 16)] = row_all[pl.ds(o + v * 16, 16)]
            ev_buf[pl.ds(v * 16, 16)] = ev_all[pl.ds(o + v * 16, 16)]
        pltpu.sync_copy(ev_buf, acc.at[row_buf], add=True)
        return carry

    lax.fori_loop(0, NWIN, win, 0)
    plsc.subcore_barrier()
    pltpu.sync_copy(acc.at[pl.ds(s * ZCH, ZCH)],
                    out_hbm.at[c, pl.ds(s * ZCH, ZCH)])


@functools.cache
def _rowsum_call():
    return pl.kernel(
        _rowsum_body,
        out_type=jax.ShapeDtypeStruct((NC, NACC), jnp.float32),
        mesh=_mesh(),
        scratch_types=[
            pltpu.VMEM((EP_TILE,), jnp.int32),
            pltpu.VMEM((EP_TILE,), jnp.float32),
            pltpu.VMEM((KW,), jnp.int32),
            pltpu.VMEM((KW,), jnp.float32),
            pltpu.VMEM_SHARED((NACC,), jnp.float32),
            pltpu.SemaphoreType.DMA,
        ],
    )


def _spmm_body(g_hbm, row_hbm, col_hbm, ev_hbm, z_hbm, out_hbm, row_buf0,
               col_buf0, ev_buf0, row_buf1, col_buf1, ev_buf1, rows0, rows1,
               acc, sem, esem0, esem1, gsem0, gsem1, ssem0, ssem1):
    c = lax.axis_index("c")
    s = lax.axis_index("s")
    wid = c * NS + s
    base = wid * EP_TILE
    row_bufs = (row_buf0, row_buf1)
    col_bufs = (col_buf0, col_buf1)
    ev_bufs = (ev_buf0, ev_buf1)
    rows = (rows0, rows1)
    esems = (esem0, esem1)
    gsems = (gsem0, gsem1)
    ssems = (ssem0, ssem1)

    zcp = pltpu.async_copy(z_hbm.at[pl.ds(s * NROW_T, NROW_T)],
                           acc.at[pl.ds(s * NROW_T, NROW_T)], sem)

    def estart(b, o):
        pltpu.async_copy(row_hbm.at[pl.ds(base + o, KW)], row_bufs[b], esems[b])
        pltpu.async_copy(col_hbm.at[pl.ds(base + o, KW)], col_bufs[b], esems[b])
        pltpu.async_copy(ev_hbm.at[pl.ds(base + o, KW)], ev_bufs[b], esems[b])

    def ewait(b, o):
        pltpu.make_async_copy(
            row_hbm.at[pl.ds(base + o, KW)], row_bufs[b], esems[b]).wait()
        pltpu.make_async_copy(
            col_hbm.at[pl.ds(base + o, KW)], col_bufs[b], esems[b]).wait()
        pltpu.make_async_copy(
            ev_hbm.at[pl.ds(base + o, KW)], ev_bufs[b], esems[b]).wait()

    def gstart(b):
        pltpu.async_copy(g_hbm.at[col_bufs[b]], rows[b], gsems[b])

    def gwait(b):
        pltpu.make_async_copy(g_hbm.at[col_bufs[b]], rows[b], gsems[b]).wait()

    def scale(b):
        def scale16(e16, carry2):
            e0 = e16 * 16
            ew16 = ev_bufs[b][pl.ds(e0, 16)]
            for j in range(16):
                bc = jnp.full((16,), ew16[j], jnp.float32)
                for f in range(D // 16):
                    rows[b][e0 + j, pl.ds(f * 16, 16)] = (
                        rows[b][e0 + j, pl.ds(f * 16, 16)] * bc)
            return carry2

        lax.fori_loop(0, KW // 16, scale16, 0)

    def sstart(b):
        pltpu.async_copy(rows[b], acc.at[row_bufs[b]], ssems[b], add=True)

    def swait(b):
        pltpu.make_async_copy(rows[b], acc.at[row_bufs[b]], ssems[b]).wait()

    def pipe(w, first, last):
        if not first:
            swait(1)
        estart(1, (w + 1) * KW)
        ewait(1, (w + 1) * KW)
        gstart(1)
        gwait(0)
        scale(0)
        sstart(0)
        gwait(1)
        scale(1)
        swait(0)
        if not last:
            estart(0, (w + 2) * KW)
            ewait(0, (w + 2) * KW)
            gstart(0)
        sstart(1)

    estart(0, 0)
    ewait(0, 0)
    gstart(0)
    zcp.wait()
    plsc.subcore_barrier()
    pipe(0, True, False)
    lax.fori_loop(1, NWIN // 2 - 1,
                  lambda k, carry: (pipe(2 * k, False, False), carry)[1], 0)
    pipe(NWIN - 2, False, True)
    swait(1)
    plsc.subcore_barrier()
    pltpu.sync_copy(acc.at[pl.ds(s * NROW_T, NROW_T)],
                    out_hbm.at[c, pl.ds(s * NROW_T, NROW_T)])


@functools.cache
def _spmm_call():
    return pl.kernel(
        _spmm_body,
        out_type=jax.ShapeDtypeStruct((NC, NP, D), jnp.float32),
        mesh=_mesh(),
        scratch_types=[
            pltpu.VMEM((KW,), jnp.int32),
            pltpu.VMEM((KW,), jnp.int32),
            pltpu.VMEM((KW,), jnp.float32),
            pltpu.VMEM((KW,), jnp.int32),
            pltpu.VMEM((KW,), jnp.int32),
            pltpu.VMEM((KW,), jnp.float32),
            pltpu.VMEM((KW, D), jnp.float32),
            pltpu.VMEM((KW, D), jnp.float32),
            pltpu.VMEM_SHARED((NP, D), jnp.float32),
            pltpu.SemaphoreType.DMA,
            pltpu.SemaphoreType.DMA,
            pltpu.SemaphoreType.DMA,
            pltpu.SemaphoreType.DMA,
            pltpu.SemaphoreType.DMA,
            pltpu.SemaphoreType.DMA,
            pltpu.SemaphoreType.DMA,
        ],
    )


def _dvec(rsp_ref):
    rs = rsp_ref[0, :] + rsp_ref[1, :] + 1e-6
    return jnp.clip(lax.rsqrt(rs), 0.0, 10.0)


def _k2_body(rsp_ref, x_ref, w0_ref, g1_ref):
    dv = _dvec(rsp_ref)
    xw = jnp.dot(x_ref[...], w0_ref[...], preferred_element_type=jnp.float32)
    g1_ref[...] = dv[:, None] * xw


def _k5_body(rsp_ref, hp_ref, w1_ref, g2_ref):
    dv = _dvec(rsp_ref)
    h = jax.nn.relu(dv[:, None] * (hp_ref[0] + hp_ref[1]))
    hw = jnp.dot(h, w1_ref[...], preferred_element_type=jnp.float32)
    g2_ref[...] = dv[:, None] * hw


def _k6_body(rsp_ref, op_ref, out_ref):
    dv = _dvec(rsp_ref)
    out_ref[...] = dv[:, None] * (op_ref[0] + op_ref[1])


_rsp_spec = pl.BlockSpec((NC, BN), lambda i: (0, i))
_mat_spec = pl.BlockSpec((BN, D), lambda i: (i, 0))
_par_spec = pl.BlockSpec((NC, BN, D), lambda i: (0, i, 0))
_w_spec = pl.BlockSpec((D, D), lambda i: (0, 0))

_k2_call = pl.pallas_call(
    _k2_body,
    grid=(NP // BN,),
    in_specs=[_rsp_spec, _mat_spec, _w_spec],
    out_specs=_mat_spec,
    out_shape=jax.ShapeDtypeStruct((NP, D), jnp.float32),
)

_k5_call = pl.pallas_call(
    _k5_body,
    grid=(NP // BN,),
    in_specs=[_rsp_spec, _par_spec, _w_spec],
    out_specs=_mat_spec,
    out_shape=jax.ShapeDtypeStruct((NP, D), jnp.float32),
)

_k6_call = pl.pallas_call(
    _k6_body,
    grid=(NP // BN,),
    in_specs=[_rsp_spec, _par_spec],
    out_specs=_mat_spec,
    out_shape=jax.ShapeDtypeStruct((NP, D), jnp.float32),
)


def kernel(x, edge_index, edge_values, W0, W1):
    row = edge_index[0]
    col = edge_index[1]
    pad = EP - E
    pad_idx = (jnp.arange(pad, dtype=jnp.int32) % N)
    row_p = jnp.concatenate([row, pad_idx])
    col_p = jnp.concatenate([col, pad_idx])
    ev_p = jnp.concatenate([edge_values, jnp.zeros((pad,), jnp.float32)])
    z1 = jnp.zeros((NACC,), jnp.float32)
    z2 = jnp.zeros((NP, D), jnp.float32)
    x_p = jnp.concatenate([x, jnp.zeros((NP - N, D), jnp.float32)])

    rsp = _rowsum_call()(row_p, ev_p, z1)
    g1 = _k2_call(rsp, x_p, W0)
    hp = _spmm_call()(g1, row_p, col_p, ev_p, z2)
    g2 = _k5_call(rsp, hp, W1)
    op = _spmm_call()(g2, row_p, col_p, ev_p, z2)
    return _k6_call(rsp, op)[:N]


# speedup vs baseline: 25.3891x; 1.3238x over previous
import functools

import jax
import jax.numpy as jnp
from jax import lax
from jax.experimental import pallas as pl
from jax.experimental.pallas import tpu as pltpu
from jax.experimental.pallas import tpu_sc as plsc

N = 10000
E = 320000
D = 128
NC = 2
NS = 16
NW = NC * NS
EP_TILE = 10240
EP = EP_TILE * NW
KW = 64
NWIN = EP_TILE // KW
NBUF = 5
NP = 10240
NACC = NP
ZCH = NACC // NS
NROW_T = NP // NS
BN = 1024

def _mesh():
    return plsc.VectorSubcoreMesh(
        core_axis_name="c", subcore_axis_name="s",
        num_cores=NC, num_subcores=NS)


def _rowsum_body(row_hbm, ev_hbm, z_hbm, out_hbm, row_all, ev_all, row_buf,
                 ev_buf, acc, sem):
    c = lax.axis_index("c")
    s = lax.axis_index("s")
    wid = c * NS + s
    base = wid * EP_TILE
    pltpu.sync_copy(z_hbm.at[pl.ds(s * ZCH, ZCH)], acc.at[pl.ds(s * ZCH, ZCH)])
    plsc.subcore_barrier()
    pltpu.async_copy(row_hbm.at[pl.ds(base, EP_TILE)], row_all, sem).wait()
    pltpu.async_copy(ev_hbm.at[pl.ds(base, EP_TILE)], ev_all, sem).wait()

    def win(w, carry):
        o = w * KW
        for v in range(KW // 16):
            row_buf[pl.ds(v * 16, 16)] = row_all[pl.ds(o + v * 16, 16)]
            ev_buf[pl.ds(v * 16, 16)] = ev_all[pl.ds(o + v * 16, 16)]
        pltpu.sync_copy(ev_buf, acc.at[row_buf], add=True)
        return carry

    lax.fori_loop(0, NWIN, win, 0)
    plsc.subcore_barrier()
    pltpu.sync_copy(acc.at[pl.ds(s * ZCH, ZCH)],
                    out_hbm.at[c, pl.ds(s * ZCH, ZCH)])


@functools.cache
def _rowsum_call():
    return pl.kernel(
        _rowsum_body,
        out_type=jax.ShapeDtypeStruct((NC, NACC), jnp.float32),
        mesh=_mesh(),
        scratch_types=[
            pltpu.VMEM((EP_TILE,), jnp.int32),
            pltpu.VMEM((EP_TILE,), jnp.float32),
            pltpu.VMEM((KW,), jnp.int32),
            pltpu.VMEM((KW,), jnp.float32),
            pltpu.VMEM_SHARED((NACC,), jnp.float32),
            pltpu.SemaphoreType.DMA,
        ],
    )


def _spmm_body(g_hbm, row_hbm, col_hbm, ev_hbm, z_hbm, out_hbm, *sc):
    row_bufs = sc[0:NBUF]
    col_bufs = sc[NBUF:2 * NBUF]
    ev_bufs = sc[2 * NBUF:3 * NBUF]
    rows = sc[3 * NBUF:4 * NBUF]
    acc = sc[4 * NBUF]
    sem = sc[4 * NBUF + 1]
    esems = sc[4 * NBUF + 2:4 * NBUF + 2 + NBUF]
    gsems = sc[4 * NBUF + 2 + NBUF:4 * NBUF + 2 + 2 * NBUF]
    ssems = sc[4 * NBUF + 2 + 2 * NBUF:4 * NBUF + 2 + 3 * NBUF]

    c = lax.axis_index("c")
    s = lax.axis_index("s")
    wid = c * NS + s
    base = wid * EP_TILE

    zcp = pltpu.async_copy(z_hbm.at[pl.ds(s * NROW_T, NROW_T)],
                           acc.at[pl.ds(s * NROW_T, NROW_T)], sem)

    def estart(b, w):
        o = base + w * KW
        pltpu.async_copy(row_hbm.at[pl.ds(o, KW)], row_bufs[b], esems[b])
        pltpu.async_copy(col_hbm.at[pl.ds(o, KW)], col_bufs[b], esems[b])
        pltpu.async_copy(ev_hbm.at[pl.ds(o, KW)], ev_bufs[b], esems[b])

    def ewait(b, w):
        o = base + w * KW
        pltpu.make_async_copy(
            row_hbm.at[pl.ds(o, KW)], row_bufs[b], esems[b]).wait()
        pltpu.make_async_copy(
            col_hbm.at[pl.ds(o, KW)], col_bufs[b], esems[b]).wait()
        pltpu.make_async_copy(
            ev_hbm.at[pl.ds(o, KW)], ev_bufs[b], esems[b]).wait()

    def gstart(b):
        pltpu.async_copy(g_hbm.at[col_bufs[b]], rows[b], gsems[b])

    def gwait(b):
        pltpu.make_async_copy(g_hbm.at[col_bufs[b]], rows[b], gsems[b]).wait()

    def scale(b):
        def scale16(e16, carry2):
            e0 = e16 * 16
            ew16 = ev_bufs[b][pl.ds(e0, 16)]
            for j in range(16):
                bc = jnp.full((16,), ew16[j], jnp.float32)
                for f in range(D // 16):
                    rows[b][e0 + j, pl.ds(f * 16, 16)] = (
                        rows[b][e0 + j, pl.ds(f * 16, 16)] * bc)
            return carry2

        lax.fori_loop(0, KW // 16, scale16, 0)

    def sstart(b):
        pltpu.async_copy(rows[b], acc.at[row_bufs[b]], ssems[b], add=True)

    def swait(b):
        pltpu.make_async_copy(rows[b], acc.at[row_bufs[b]], ssems[b]).wait()

    def win_ops(w, b):
        bp = (b + 3) % NBUF
        bg = (b + 2) % NBUF

        @pl.when(w >= 2)
        def _():
            swait(bp)

        @pl.when(w <= NWIN - 4)
        def _():
            estart(bp, w + 3)

        @pl.when(w <= NWIN - 3)
        def _():
            ewait(bg, w + 2)
            gstart(bg)

        gwait(b)
        scale(b)
        sstart(b)

    estart(0, 0)
    estart(1, 1)
    estart(2, 2)
    ewait(0, 0)
    gstart(0)
    ewait(1, 1)
    gstart(1)
    zcp.wait()
    plsc.subcore_barrier()
    lax.fori_loop(
        0, NWIN // NBUF,
        lambda k, carry: ([win_ops(NBUF * k + j, j) for j in range(NBUF)],
                          carry)[1], 0)
    swait((NWIN - 2) % NBUF)
    swait((NWIN - 1) % NBUF)
    plsc.subcore_barrier()
    pltpu.sync_copy(acc.at[pl.ds(s * NROW_T, NROW_T)],
                    out_hbm.at[c, pl.ds(s * NROW_T, NROW_T)])


@functools.cache
def _spmm_call():
    return pl.kernel(
        _spmm_body,
        out_type=jax.ShapeDtypeStruct((NC, NP, D), jnp.float32),
        mesh=_mesh(),
        scratch_types=(
            [pltpu.VMEM((KW,), jnp.int32) for _ in range(NBUF)]
            + [pltpu.VMEM((KW,), jnp.int32) for _ in range(NBUF)]
            + [pltpu.VMEM((KW,), jnp.float32) for _ in range(NBUF)]
            + [pltpu.VMEM((KW, D), jnp.float32) for _ in range(NBUF)]
            + [pltpu.VMEM_SHARED((NP, D), jnp.float32)]
            + [pltpu.SemaphoreType.DMA] * (1 + 3 * NBUF)
        ),
    )


def _dvec(rsp_ref):
    rs = rsp_ref[0, :] + rsp_ref[1, :] + 1e-6
    return jnp.clip(lax.rsqrt(rs), 0.0, 10.0)


def _k2_body(rsp_ref, x_ref, w0_ref, g1_ref):
    dv = _dvec(rsp_ref)
    xw = jnp.dot(x_ref[...], w0_ref[...], preferred_element_type=jnp.float32)
    g1_ref[...] = dv[:, None] * xw


def _k5_body(rsp_ref, hp_ref, w1_ref, g2_ref):
    dv = _dvec(rsp_ref)
    h = jax.nn.relu(dv[:, None] * (hp_ref[0] + hp_ref[1]))
    hw = jnp.dot(h, w1_ref[...], preferred_element_type=jnp.float32)
    g2_ref[...] = dv[:, None] * hw


def _k6_body(rsp_ref, op_ref, out_ref):
    dv = _dvec(rsp_ref)
    out_ref[...] = dv[:, None] * (op_ref[0] + op_ref[1])


_rsp_spec = pl.BlockSpec((NC, BN), lambda i: (0, i))
_mat_spec = pl.BlockSpec((BN, D), lambda i: (i, 0))
_par_spec = pl.BlockSpec((NC, BN, D), lambda i: (0, i, 0))
_w_spec = pl.BlockSpec((D, D), lambda i: (0, 0))

_k2_call = pl.pallas_call(
    _k2_body,
    grid=(NP // BN,),
    in_specs=[_rsp_spec, _mat_spec, _w_spec],
    out_specs=_mat_spec,
    out_shape=jax.ShapeDtypeStruct((NP, D), jnp.float32),
)

_k5_call = pl.pallas_call(
    _k5_body,
    grid=(NP // BN,),
    in_specs=[_rsp_spec, _par_spec, _w_spec],
    out_specs=_mat_spec,
    out_shape=jax.ShapeDtypeStruct((NP, D), jnp.float32),
)

_k6_call = pl.pallas_call(
    _k6_body,
    grid=(NP // BN,),
    in_specs=[_rsp_spec, _par_spec],
    out_specs=_mat_spec,
    out_shape=jax.ShapeDtypeStruct((NP, D), jnp.float32),
)


def kernel(x, edge_index, edge_values, W0, W1):
    row = edge_index[0]
    col = edge_index[1]
    pad = EP - E
    pad_idx = (jnp.arange(pad, dtype=jnp.int32) % N)
    row_p = jnp.concatenate([row, pad_idx])
    col_p = jnp.concatenate([col, pad_idx])
    ev_p = jnp.concatenate([edge_values, jnp.zeros((pad,), jnp.float32)])
    z1 = jnp.zeros((NACC,), jnp.float32)
    z2 = jnp.zeros((NP, D), jnp.float32)
    x_p = jnp.concatenate([x, jnp.zeros((NP - N, D), jnp.float32)])

    rsp = _rowsum_call()(row_p, ev_p, z1)
    g1 = _k2_call(rsp, x_p, W0)
    hp = _spmm_call()(g1, row_p, col_p, ev_p, z2)
    g2 = _k5_call(rsp, hp, W1)
    op = _spmm_call()(g2, row_p, col_p, ev_p, z2)
    return _k6_call(rsp, op)[:N]


# speedup vs baseline: 25.6089x; 1.0087x over previous
import functools

import jax
import jax.numpy as jnp
from jax import lax
from jax.experimental import pallas as pl
from jax.experimental.pallas import tpu as pltpu
from jax.experimental.pallas import tpu_sc as plsc

N = 10000
E = 320000
D = 128
NC = 2
NS = 16
NW = NC * NS
EP_TILE = 10240
EP = EP_TILE * NW
KW = 64
NWIN = EP_TILE // KW
NBUF = 5
NP = 10240
NACC = NP
ZCH = NACC // NS
NROW_T = NP // NS
BN = 1024

def _mesh():
    return plsc.VectorSubcoreMesh(
        core_axis_name="c", subcore_axis_name="s",
        num_cores=NC, num_subcores=NS)


KW1 = 128
NWIN1 = EP_TILE // KW1
NB1 = 4


def _rowsum_body(epk_hbm, ev_hbm, z_hbm, out_hbm, *sc):
    ebufs = sc[0:NB1]
    evfs = sc[NB1:2 * NB1]
    acc = sc[2 * NB1]
    sem = sc[2 * NB1 + 1]
    esems = sc[2 * NB1 + 2:2 * NB1 + 2 + NB1]
    ssems = sc[2 * NB1 + 2 + NB1:2 * NB1 + 2 + 2 * NB1]

    c = lax.axis_index("c")
    s = lax.axis_index("s")
    wid = c * NS + s
    base = wid * EP_TILE

    zcp = pltpu.async_copy(z_hbm.at[pl.ds(s * ZCH, ZCH)],
                           acc.at[pl.ds(s * ZCH, ZCH)], sem)

    def estart(b, w):
        pltpu.async_copy(epk_hbm.at[:, pl.ds(base + w * KW1, KW1)],
                         ebufs[b], esems[b])
        pltpu.async_copy(ev_hbm.at[pl.ds(base + w * KW1, KW1)],
                         evfs[b], esems[b])

    def ewait(b, w):
        pltpu.make_async_copy(epk_hbm.at[:, pl.ds(base + w * KW1, KW1)],
                              ebufs[b], esems[b]).wait()
        pltpu.make_async_copy(ev_hbm.at[pl.ds(base + w * KW1, KW1)],
                              evfs[b], esems[b]).wait()

    def sstart(b):
        pltpu.async_copy(evfs[b], acc.at[ebufs[b].at[0]], ssems[b], add=True)

    def swait(b):
        pltpu.make_async_copy(evfs[b], acc.at[ebufs[b].at[0]], ssems[b]).wait()

    def win_ops(w, b):
        @pl.when(w >= 2)
        def _():
            swait((b + 2) % NB1)

        @pl.when(w <= NWIN1 - 3)
        def _():
            estart((b + 2) % NB1, w + 2)

        ewait(b, w)
        sstart(b)

    estart(0, 0)
    estart(1, 1)
    zcp.wait()
    plsc.subcore_barrier()
    lax.fori_loop(
        0, NWIN1 // NB1,
        lambda k, carry: ([win_ops(NB1 * k + j, j) for j in range(NB1)],
                          carry)[1], 0)
    swait((NWIN1 - 2) % NB1)
    swait((NWIN1 - 1) % NB1)
    plsc.subcore_barrier()
    pltpu.sync_copy(acc.at[pl.ds(s * ZCH, ZCH)],
                    out_hbm.at[c, pl.ds(s * ZCH, ZCH)])


@functools.cache
def _rowsum_call():
    return pl.kernel(
        _rowsum_body,
        out_type=jax.ShapeDtypeStruct((NC, NACC), jnp.float32),
        mesh=_mesh(),
        scratch_types=(
            [pltpu.VMEM((2, KW1), jnp.int32) for _ in range(NB1)]
            + [pltpu.VMEM((KW1,), jnp.float32) for _ in range(NB1)]
            + [pltpu.VMEM_SHARED((NACC,), jnp.float32)]
            + [pltpu.SemaphoreType.DMA] * (1 + 2 * NB1)
        ),
    )


def _spmm_body(g_hbm, row_hbm, col_hbm, ev_hbm, z_hbm, out_hbm, *sc):
    rowbufs = sc[0:NBUF]
    colbufs = sc[NBUF:2 * NBUF]
    evbufs = sc[2 * NBUF:3 * NBUF]
    rows = sc[3 * NBUF:4 * NBUF]
    acc = sc[4 * NBUF]
    sem = sc[4 * NBUF + 1]
    esems = sc[4 * NBUF + 2:4 * NBUF + 2 + NBUF]
    gsems = sc[4 * NBUF + 2 + NBUF:4 * NBUF + 2 + 2 * NBUF]
    ssems = sc[4 * NBUF + 2 + 2 * NBUF:4 * NBUF + 2 + 3 * NBUF]

    c = lax.axis_index("c")
    s = lax.axis_index("s")
    wid = c * NS + s
    base = wid * EP_TILE

    zcp = pltpu.async_copy(z_hbm.at[pl.ds(s * NROW_T, NROW_T)],
                           acc.at[pl.ds(s * NROW_T, NROW_T)], sem)

    def estart(b, w):
        o = base + w * KW
        pltpu.async_copy(row_hbm.at[pl.ds(o, KW)], rowbufs[b], esems[b])
        pltpu.async_copy(col_hbm.at[pl.ds(o, KW)], colbufs[b], esems[b])
        pltpu.async_copy(ev_hbm.at[pl.ds(o, KW)], evbufs[b], esems[b])

    def ewait(b, w):
        o = base + w * KW
        pltpu.make_async_copy(
            row_hbm.at[pl.ds(o, KW)], rowbufs[b], esems[b]).wait()
        pltpu.make_async_copy(
            col_hbm.at[pl.ds(o, KW)], colbufs[b], esems[b]).wait()
        pltpu.make_async_copy(
            ev_hbm.at[pl.ds(o, KW)], evbufs[b], esems[b]).wait()

    def gstart(b):
        pltpu.async_copy(g_hbm.at[colbufs[b]], rows[b], gsems[b])

    def gwait(b):
        pltpu.make_async_copy(g_hbm.at[colbufs[b]], rows[b], gsems[b]).wait()

    def scale(b):
        def scale16(e16, carry2):
            e0 = e16 * 16
            ew16 = evbufs[b][pl.ds(e0, 16)]
            for j in range(16):
                bc = jnp.full((16,), ew16[j], jnp.float32)
                for f in range(D // 16):
                    rows[b][e0 + j, pl.ds(f * 16, 16)] = (
                        rows[b][e0 + j, pl.ds(f * 16, 16)] * bc)
            return carry2

        lax.fori_loop(0, KW // 16, scale16, 0)

    def sstart(b):
        pltpu.async_copy(rows[b], acc.at[rowbufs[b]], ssems[b], add=True)

    def swait(b):
        pltpu.make_async_copy(rows[b], acc.at[rowbufs[b]], ssems[b]).wait()

    def win_ops(w, b):
        bp = (b + 3) % NBUF
        bg = (b + 2) % NBUF

        @pl.when(w >= 2)
        def _():
            swait(bp)

        @pl.when(w <= NWIN - 4)
        def _():
            estart(bp, w + 3)

        @pl.when(w <= NWIN - 3)
        def _():
            ewait(bg, w + 2)
            gstart(bg)

        gwait(b)
        scale(b)
        sstart(b)

    estart(0, 0)
    estart(1, 1)
    estart(2, 2)
    ewait(0, 0)
    gstart(0)
    ewait(1, 1)
    gstart(1)
    zcp.wait()
    plsc.subcore_barrier()
    lax.fori_loop(
        0, NWIN // NBUF,
        lambda k, carry: ([win_ops(NBUF * k + j, j) for j in range(NBUF)],
                          carry)[1], 0)
    swait((NWIN - 2) % NBUF)
    swait((NWIN - 1) % NBUF)
    plsc.subcore_barrier()
    pltpu.sync_copy(acc.at[pl.ds(s * NROW_T, NROW_T)],
                    out_hbm.at[c, pl.ds(s * NROW_T, NROW_T)])


@functools.cache
def _spmm_call():
    return pl.kernel(
        _spmm_body,
        out_type=jax.ShapeDtypeStruct((NC, NP, D), jnp.float32),
        mesh=_mesh(),
        scratch_types=(
            [pltpu.VMEM((KW,), jnp.int32) for _ in range(NBUF)]
            + [pltpu.VMEM((KW,), jnp.int32) for _ in range(NBUF)]
            + [pltpu.VMEM((KW,), jnp.float32) for _ in range(NBUF)]
            + [pltpu.VMEM((KW, D), jnp.float32) for _ in range(NBUF)]
            + [pltpu.VMEM_SHARED((NP, D), jnp.float32)]
            + [pltpu.SemaphoreType.DMA] * (1 + 3 * NBUF)
        ),
    )


def _dvec(rsp_ref):
    rs = rsp_ref[0, :] + rsp_ref[1, :] + 1e-6
    return jnp.clip(lax.rsqrt(rs), 0.0, 10.0)


def _k2_body(rsp_ref, x_ref, w0_ref, g1_ref):
    dv = _dvec(rsp_ref)
    xw = jnp.dot(x_ref[...], w0_ref[...], preferred_element_type=jnp.float32)
    g1_ref[...] = dv[:, None] * xw


def _k5_body(rsp_ref, hp_ref, w1_ref, g2_ref):
    dv = _dvec(rsp_ref)
    h = jax.nn.relu(dv[:, None] * (hp_ref[0] + hp_ref[1]))
    hw = jnp.dot(h, w1_ref[...], preferred_element_type=jnp.float32)
    g2_ref[...] = dv[:, None] * hw


def _k6_body(rsp_ref, op_ref, out_ref):
    dv = _dvec(rsp_ref)
    out_ref[...] = dv[:, None] * (op_ref[0] + op_ref[1])


_rsp_spec = pl.BlockSpec((NC, BN), lambda i: (0, i))
_mat_spec = pl.BlockSpec((BN, D), lambda i: (i, 0))
_par_spec = pl.BlockSpec((NC, BN, D), lambda i: (0, i, 0))
_w_spec = pl.BlockSpec((D, D), lambda i: (0, 0))

_k2_call = pl.pallas_call(
    _k2_body,
    grid=(NP // BN,),
    in_specs=[_rsp_spec, _mat_spec, _w_spec],
    out_specs=_mat_spec,
    out_shape=jax.ShapeDtypeStruct((NP, D), jnp.float32),
)

_k5_call = pl.pallas_call(
    _k5_body,
    grid=(NP // BN,),
    in_specs=[_rsp_spec, _par_spec, _w_spec],
    out_specs=_mat_spec,
    out_shape=jax.ShapeDtypeStruct((NP, D), jnp.float32),
)

_k6_call = pl.pallas_call(
    _k6_body,
    grid=(NP // BN,),
    in_specs=[_rsp_spec, _par_spec],
    out_specs=_mat_spec,
    out_shape=jax.ShapeDtypeStruct((NP, D), jnp.float32),
)


def kernel(x, edge_index, edge_values, W0, W1):
    row = edge_index[0]
    col = edge_index[1]
    pad = EP - E
    pad_idx = (jnp.arange(pad, dtype=jnp.int32) % N)
    row_p = jnp.concatenate([row, pad_idx])
    col_p = jnp.concatenate([col, pad_idx])
    ev_p = jnp.concatenate([edge_values, jnp.zeros((pad,), jnp.float32)])
    epk = jnp.stack([row_p, col_p])
    z1 = jnp.zeros((NACC,), jnp.float32)
    z2 = jnp.zeros((NP, D), jnp.float32)
    x_p = jnp.concatenate([x, jnp.zeros((NP - N, D), jnp.float32)])

    rsp = _rowsum_call()(epk, ev_p, z1)
    g1 = _k2_call(rsp, x_p, W0)
    hp = _spmm_call()(g1, row_p, col_p, ev_p, z2)
    g2 = _k5_call(rsp, hp, W1)
    op = _spmm_call()(g2, row_p, col_p, ev_p, z2)
    return _k6_call(rsp, op)[:N]
